# baseline (speedup 1.0000x reference)
import jax
import jax.numpy as jnp
from jax.experimental import pallas as pl

_NUM_CLASSES = 15
_THRESH = 0.05
_TOPK = 2000
_N = 384 * 384


def _mean_body(x_ref, o_ref):
    s = jax.nn.sigmoid(x_ref[...])
    o_ref[...] = jnp.mean(s, axis=0)


def _scores_mean_cm(cls_score):
    x = cls_score.reshape(8, _NUM_CLASSES, 1152, 128)
    grid = (_NUM_CLASSES, 3)
    return pl.pallas_call(
        _mean_body,
        grid=grid,
        in_specs=[pl.BlockSpec((8, 1, 384, 128), lambda c, r: (0, c, r, 0))],
        out_specs=pl.BlockSpec((1, 384, 128), lambda c, r: (c, r, 0)),
        out_shape=jax.ShapeDtypeStruct((_NUM_CLASSES, 1152, 128), jnp.float32),
    )(x)


def kernel(cls_score):
    sm_cm = _scores_mean_cm(cls_score)
    scores_mean = sm_cm.reshape(_NUM_CLASSES, _N).T

    flat = scores_mean.reshape(-1)
    valid_mask = flat > _THRESH
    num_valid = valid_mask.sum()
    num_topk = jnp.minimum(_TOPK, num_valid)
    neg = jnp.where(valid_mask, -flat, jnp.inf)
    order = jnp.argsort(neg, stable=True)
    topk_flat = order[:_TOPK]
    pos_ok = jnp.arange(_TOPK) < num_topk
    keep_idxs = topk_flat // _NUM_CLASSES

    n_rows = _N
    row_keep = (
        jnp.zeros(n_rows + 1, dtype=bool)
        .at[jnp.where(pos_ok, keep_idxs, n_rows)]
        .set(True)[:n_rows]
    )
    unique_rows = jnp.nonzero(row_keep, size=_TOPK, fill_value=0)[0]
    unique_count = row_keep.sum()
    u_ok = jnp.arange(_TOPK) < unique_count
    sm = scores_mean[unique_rows]
    similar = sm[:, jnp.array([3, 5])]
    lab_is_x = similar[:, 0] > similar[:, 1]
    m_x = (u_ok & lab_is_x).astype(similar.dtype)
    m_y = (u_ok & ~lab_is_x).astype(similar.dtype)

    def _slope(x, y, m):
        n = m.sum()
        mx = (x * m).sum() / n
        my = (y * m).sum() / n
        dx = (x - mx) * m
        return (dx * (y - my)).sum() / (dx * dx).sum()

    slope_x = _slope(similar[:, 0], similar[:, 1], m_x)
    slope_y = _slope(similar[:, 0], similar[:, 1], m_y)
    TINY = 1e-05
    angle = jnp.arctan(jnp.abs((slope_y - slope_x) / (1 + slope_y * slope_x + TINY)))
    sca = jnp.degrees(angle)
    has_x = (u_ok & lab_is_x).any()
    cond = (num_topk > 0) & has_x
    return jnp.where(cond, sca, 0.0).astype(jnp.float32)


# speedup vs baseline: 8.2243x; 8.2243x over previous
import functools

import jax
import jax.numpy as jnp
from jax import lax
from jax.experimental import pallas as pl
from jax.experimental.pallas import tpu as pltpu
from jax.experimental.pallas import tpu_sc as plsc

_NUM_CLASSES = 15
_THRESH = 0.05
_TOPK = 2000
_N = 384 * 384
_TOT = _N * _NUM_CLASSES

_NW = 32
_CHUNK = _TOT // _NW
_NVEC = _CHUNK // 16
_HBINS = 2048
_H2BINS = 32768
_FBINS = 2048
_CAP = 8192
_BIAS = 0x3D000000
_BIG_FLAT = _TOT + 1


def _mean_body(x_ref, o_ref):
    s = jax.nn.sigmoid(x_ref[...])
    o_ref[...] = jnp.mean(s, axis=0)


def _scores_mean_cm(cls_score):
    x = cls_score.reshape(8, _NUM_CLASSES, 1152, 128)
    return pl.pallas_call(
        _mean_body,
        grid=(_NUM_CLASSES, 3),
        in_specs=[pl.BlockSpec((8, 1, 384, 128), lambda c, r: (0, c, r, 0))],
        out_specs=pl.BlockSpec((1, 384, 128), lambda c, r: (c, r, 0)),
        out_shape=jax.ShapeDtypeStruct((_NUM_CLASSES, 1152, 128), jnp.float32),
    )(x)


def _zero_vmem(ref, nwords):
    z = jnp.zeros((16,), jnp.int32)

    def z_body(i, _):
        ref[pl.ds(i * 16, 16)] = z
        return 0

    lax.fori_loop(0, nwords // 16, z_body, 0)


def _scan_top(hist_ref, nbins, base, target):
    iota = lax.iota(jnp.int32, 16)
    neg = jnp.int32(-2147483647)

    def body(j, carry):
        found, b, cabove, cnt = carry
        c = nbins // 16 - 1 - j
        h = hist_ref[pl.ds(c * 16, 16)]
        cum = plsc.cumsum(h)
        s = jnp.max(cum)
        ge = (cnt + s) - (cum - h)
        mask = ge >= target
        hit = jnp.logical_and(jnp.logical_not(found), (cnt + s) >= target)
        l_ = jnp.max(jnp.where(mask, iota, -1))
        cum_at = jnp.max(jnp.where(iota == l_, cum, neg))
        nb = c * 16 + l_
        ncab = cnt + s - cum_at
        found2 = jnp.logical_or(found, hit)
        b2 = jnp.where(hit, nb, b)
        cab2 = jnp.where(hit, ncab, cabove)
        return found2, b2, cab2, cnt + s

    return lax.fori_loop(
        0, nbins // 16, body,
        (jnp.bool_(False), jnp.int32(-1), jnp.int32(0), base))


def _scan_bottom(hist_ref, nbins, target):
    iota = lax.iota(jnp.int32, 16)
    neg = jnp.int32(-2147483647)

    def body(j, carry):
        found, b, cbelow, cnt = carry
        h = hist_ref[pl.ds(j * 16, 16)]
        cum = plsc.cumsum(h)
        s = jnp.max(cum)
        le = cnt + cum
        mask = le >= target
        hit = jnp.logical_and(jnp.logical_not(found), (cnt + s) >= target)
        l_ = jnp.min(jnp.where(mask, iota, 99))
        cum_at = jnp.max(jnp.where(iota == l_, cum, neg))
        h_at = jnp.max(jnp.where(iota == l_, h, neg))
        nb = j * 16 + l_
        ncb = cnt + cum_at - h_at
        found2 = jnp.logical_or(found, hit)
        b2 = jnp.where(hit, nb, b)
        cb2 = jnp.where(hit, ncb, cbelow)
        return found2, b2, cb2, cnt + s

    res = lax.fori_loop(
        0, nbins // 16, body,
        (jnp.bool_(False), jnp.int32(0), jnp.int32(0), jnp.int32(0)))
    return res[1], res[2]


def _merge_hist(hist_hbm_refs, hbuf, H):
    pltpu.sync_copy(hist_hbm_refs, hbuf)

    def m_body(i, _):
        H[pl.ds(i * 16, 16)] = hbuf[0, pl.ds(i * 16, 16)] + hbuf[1, pl.ds(i * 16, 16)]
        return 0

    lax.fori_loop(0, _HBINS // 16, m_body, 0)


_sc_mesh = plsc.VectorSubcoreMesh(core_axis_name="c", subcore_axis_name="s")


@functools.partial(
    pl.kernel,
    out_type=jax.ShapeDtypeStruct((2, _HBINS), jnp.int32),
    mesh=_sc_mesh,
    compiler_params=pltpu.CompilerParams(needs_layout_passes=False),
    scratch_types=[
        pltpu.VMEM((_CHUNK,), jnp.float32),
        pltpu.VMEM((_HBINS,), jnp.int32),
        pltpu.VMEM_SHARED((16, _HBINS), jnp.int32),
        pltpu.VMEM((16, _HBINS), jnp.int32),
    ],
)
def _sc_hist1(smf_hbm, out_hbm, vals, hist, shared, redbuf):
    cid = lax.axis_index("c")
    sid = lax.axis_index("s")
    wid = sid * 2 + cid
    pltpu.sync_copy(smf_hbm.at[pl.ds(wid * _CHUNK, _CHUNK)], vals)
    _zero_vmem(hist, _HBINS)
    ones = jnp.ones((16,), jnp.int32)

    def body(i, _):
        v = vals[pl.ds(i * 16, 16)]
        b = lax.bitcast_convert_type(v, jnp.int32)
        valid = v > _THRESH
        bin_ = lax.shift_right_logical(b - _BIAS, 15)
        bin_ = jnp.where(valid, bin_, 0)
        plsc.addupdate_scatter(hist, [bin_], ones, mask=valid)
        return 0

    lax.fori_loop(0, _NVEC, body, 0)
    pltpu.sync_copy(hist, shared.at[sid])
    plsc.subcore_barrier()

    @pl.when(sid == 0)
    def _():
        pltpu.sync_copy(shared, redbuf)

        def red(i, _):
            acc = redbuf[0, pl.ds(i * 16, 16)]
            for t in range(1, 16):
                acc = acc + redbuf[t, pl.ds(i * 16, 16)]
            hist[pl.ds(i * 16, 16)] = acc
            return 0

        lax.fori_loop(0, _HBINS // 16, red, 0)
        pltpu.sync_copy(hist, out_hbm.at[cid])


@functools.partial(
    pl.kernel,
    out_type=(
        jax.ShapeDtypeStruct((_NW * _CAP,), jnp.int32),
        jax.ShapeDtypeStruct((_NW * _CAP,), jnp.int32),
        jax.ShapeDtypeStruct((_NW * 16,), jnp.int32),
    ),
    mesh=_sc_mesh,
    compiler_params=pltpu.CompilerParams(needs_layout_passes=False),
    scratch_types=[
        pltpu.VMEM((_CHUNK,), jnp.float32),
        pltpu.VMEM((2, _HBINS), jnp.int32),
        pltpu.VMEM((_HBINS,), jnp.int32),
        pltpu.VMEM((_CAP,), jnp.int32),
        pltpu.VMEM((_CAP,), jnp.int32),
        pltpu.VMEM((16,), jnp.int32),
    ],
)
def _sc_compact(smf_hbm, hist_hbm, ck_hbm, cf_hbm, cnt_hbm,
                vals, hbuf, H, ck, cf, cntv):
    cid = lax.axis_index("c")
    sid = lax.axis_index("s")
    wid = sid * 2 + cid
    pltpu.sync_copy(smf_hbm.at[pl.ds(wid * _CHUNK, _CHUNK)], vals)
    _merge_hist(hist_hbm, hbuf, H)
    found, b1, _, _ = _scan_top(H, _HBINS, jnp.int32(0), jnp.int32(_TOPK))
    b1 = jnp.where(found, b1, jnp.int32(-1))
    iota = lax.iota(jnp.int32, 16)

    def body(i, carry):
        off, jv = carry
        v = vals[pl.ds(i * 16, 16)]
        b = lax.bitcast_convert_type(v, jnp.int32)
        valid = v > _THRESH
        key = b - _BIAS
        m = jnp.logical_and(valid, lax.shift_right_logical(key, 15) == b1)
        incl = plsc.cumsum(m.astype(jnp.int32))
        tot = jnp.max(incl)
        cdiv = jv // _N
        flat = (jv - cdiv * _N) * _NUM_CLASSES + cdiv
        pos = off + incl - 1
        m2 = jnp.logical_and(m, pos < _CAP)
        pos = jnp.where(m2, pos, 0)
        plsc.store_scatter(ck, [pos], key, mask=m2)
        plsc.store_scatter(cf, [pos], flat, mask=m2)
        return off + tot, jv + 16

    off0 = jnp.int32(0)
    jv0 = wid * _CHUNK + iota
    off_end, _ = lax.fori_loop(0, _NVEC, body, (off0, jv0))
    cntv[...] = jnp.where(iota == 0, jnp.minimum(off_end, _CAP), 0)
    pltpu.sync_copy(ck, ck_hbm.at[pl.ds(wid * _CAP, _CAP)])
    pltpu.sync_copy(cf, cf_hbm.at[pl.ds(wid * _CAP, _CAP)])
    pltpu.sync_copy(cntv, cnt_hbm.at[pl.ds(wid * 16, 16)])


@functools.partial(
    pl.kernel,
    out_type=(
        jax.ShapeDtypeStruct((16,), jnp.int32),
        jax.ShapeDtypeStruct((16,), jnp.float32),
    ),
    mesh=_sc_mesh,
    compiler_params=pltpu.CompilerParams(needs_layout_passes=False),
    scratch_types=[
        pltpu.VMEM((2, _HBINS), jnp.int32),
        pltpu.VMEM((_HBINS,), jnp.int32),
        pltpu.VMEM((_H2BINS,), jnp.int32),
        pltpu.VMEM((_FBINS,), jnp.int32),
        pltpu.VMEM((_CAP,), jnp.int32),
        pltpu.VMEM((_CAP,), jnp.int32),
        pltpu.VMEM((_NW * 16,), jnp.int32),
        pltpu.VMEM((16,), jnp.int32),
        pltpu.VMEM((16,), jnp.float32),
    ],
)
def _sc_resolve(hist_hbm, ck_hbm, cf_hbm, cnt_hbm, zeros_hbm,
                mi_hbm, mf_hbm,
                hbuf, H, hist2, histf, rowk, rowf, cnts, miv, mfv):
    cid = lax.axis_index("c")
    sid = lax.axis_index("s")
    wid = sid * 2 + cid

    @pl.when(wid == 0)
    def _():
        _merge_hist(hist_hbm, hbuf, H)
        found, b1, cabove1, num_valid = _scan_top(
            H, _HBINS, jnp.int32(0), jnp.int32(_TOPK))
        pltpu.sync_copy(cnt_hbm, cnts)
        pltpu.sync_copy(zeros_hbm, hist2)
        ones = jnp.ones((16,), jnp.int32)
        iota = lax.iota(jnp.int32, 16)

        def t_body2(t, _):
            cnt_t = cnts[pl.ds(t * 16, 16)][0]
            pltpu.sync_copy(ck_hbm.at[pl.ds(t * _CAP, _CAP)], rowk)

            def i_body(i, _):
                k = rowk[pl.ds(i * 16, 16)]
                msk = (i * 16 + iota) < cnt_t
                bin_ = jnp.where(msk, jnp.bitwise_and(k, 0x7FFF), 0)
                plsc.addupdate_scatter(hist2, [bin_], ones, mask=msk)
                return 0

            lax.fori_loop(0, (cnt_t + 15) // 16, i_body, 0)
            return 0

        lax.fori_loop(0, _NW, t_body2, 0)
        _, b2, count_gt, _ = _scan_top(hist2, _H2BINS, cabove1, jnp.int32(_TOPK))
        v26 = b1 * 32768 + jnp.maximum(b2, 0)
        r = jnp.int32(_TOPK) - count_gt

        pltpu.sync_copy(zeros_hbm.at[pl.ds(0, _FBINS)], histf)

        def t_bodyf1(t, _):
            cnt_t = cnts[pl.ds(t * 16, 16)][0]
            pltpu.sync_copy(ck_hbm.at[pl.ds(t * _CAP, _CAP)], rowk)
            pltpu.sync_copy(cf_hbm.at[pl.ds(t * _CAP, _CAP)], rowf)

            def i_body(i, _):
                k = rowk[pl.ds(i * 16, 16)]
                f = rowf[pl.ds(i * 16, 16)]
                msk = jnp.logical_and((i * 16 + iota) < cnt_t, k == v26)
                bin_ = jnp.where(msk, lax.shift_right_logical(f, 11), 0)
                plsc.addupdate_scatter(histf, [bin_], ones, mask=msk)
                return 0

            lax.fori_loop(0, (cnt_t + 15) // 16, i_body, 0)
            return 0

        lax.fori_loop(0, _NW, t_bodyf1, 0)
        bf1, cbelow1 = _scan_bottom(histf, _FBINS, r)
        r2 = r - cbelow1

        pltpu.sync_copy(zeros_hbm.at[pl.ds(0, _FBINS)], histf)

        def t_bodyf2(t, _):
            cnt_t = cnts[pl.ds(t * 16, 16)][0]
            pltpu.sync_copy(ck_hbm.at[pl.ds(t * _CAP, _CAP)], rowk)
            pltpu.sync_copy(cf_hbm.at[pl.ds(t * _CAP, _CAP)], rowf)

            def i_body(i, _):
                k = rowk[pl.ds(i * 16, 16)]
                f = rowf[pl.ds(i * 16, 16)]
                msk = jnp.logical_and((i * 16 + iota) < cnt_t, k == v26)
                msk = jnp.logical_and(msk, lax.shift_right_logical(f, 11) == bf1)
                bin_ = jnp.where(msk, jnp.bitwise_and(f, 0x7FF), 0)
                plsc.addupdate_scatter(histf, [bin_], ones, mask=msk)
                return 0

            lax.fori_loop(0, (cnt_t + 15) // 16, i_body, 0)
            return 0

        lax.fori_loop(0, _NW, t_bodyf2, 0)
        bf2, _ = _scan_bottom(histf, _FBINS, r2)
        istar = bf1 * 2048 + bf2

        thresh_bits = jnp.int32(0x3D4CCCCD)
        vbits = jnp.where(found, v26 + _BIAS, thresh_bits)
        istar = jnp.where(found, istar, jnp.int32(-1))
        valid_any = (num_valid > 0).astype(jnp.int32)

        mi = (jnp.where(iota == 0, istar, 0)
              + jnp.where(iota == 1, valid_any, 0)
              + jnp.where(iota == 2, found.astype(jnp.int32), 0)
              + jnp.where(iota == 3, num_valid, 0))
        miv[...] = mi
        mfv[...] = lax.bitcast_convert_type(jnp.broadcast_to(vbits, (16,)), jnp.float32)
        pltpu.sync_copy(miv, mi_hbm)
        pltpu.sync_copy(mfv, mf_hbm)


def _reg_body(mi_ref, mf_ref, x_ref, o_ref, acc_ref):
    r = pl.program_id(0)

    @pl.when(r == 0)
    def _():
        for i in range(12):
            acc_ref[i] = 0.0

    V = mf_ref[0]
    istar = mi_ref[0]
    blk = x_ref[...]
    row2d = lax.broadcasted_iota(jnp.int32, (128, 128), 0)
    lane2d = lax.broadcasted_iota(jnp.int32, (128, 128), 1)
    n = (r * 128 + row2d) * 128 + lane2d
    keep = None
    for c in range(_NUM_CLASSES):
        v = blk[c]
        flat = n * _NUM_CLASSES + c
        sel = jnp.logical_or(
            v > V, jnp.logical_and(v == V, flat <= istar))
        keep = sel if keep is None else jnp.logical_or(keep, sel)
    x = blk[3]
    y = blk[5]
    lab = x > y
    fx = jnp.logical_and(keep, lab).astype(jnp.float32)
    fy = jnp.logical_and(keep, jnp.logical_not(lab)).astype(jnp.float32)
    acc_ref[0] += jnp.sum(fx)
    acc_ref[1] += jnp.sum(fx * x)
    acc_ref[2] += jnp.sum(fx * y)
    acc_ref[3] += jnp.sum(fx * x * x)
    acc_ref[4] += jnp.sum(fx * x * y)
    acc_ref[5] += jnp.sum(fy)
    acc_ref[6] += jnp.sum(fy * x)
    acc_ref[7] += jnp.sum(fy * y)
    acc_ref[8] += jnp.sum(fy * x * x)
    acc_ref[9] += jnp.sum(fy * x * y)

    @pl.when(r == 8)
    def _():
        nX = acc_ref[0]
        sxX, syX, sxxX, sxyX = acc_ref[1], acc_ref[2], acc_ref[3], acc_ref[4]
        nY = acc_ref[5]
        sxY, syY, sxxY, sxyY = acc_ref[6], acc_ref[7], acc_ref[8], acc_ref[9]
        slope_x = (sxyX - sxX * syX / nX) / (sxxX - sxX * sxX / nX)
        slope_y = (sxyY - sxY * syY / nY) / (sxxY - sxY * sxY / nY)
        t = jnp.abs((slope_y - slope_x) / (1.0 + slope_y * slope_x + 1e-05))
        tv = jnp.full((8, 128), t)
        hi = tv > 2.414213562373095
        mid = tv > 0.414213562373095
        yofs = jnp.where(hi, jnp.float32(1.5707963267948966),
                         jnp.where(mid, jnp.float32(0.7853981633974483), 0.0))
        z = jnp.where(hi, -1.0 / tv,
                      jnp.where(mid, (tv - 1.0) / (tv + 1.0), tv))
        z2 = z * z
        p = (((8.05374449538e-2 * z2 - 1.38776856032e-1) * z2
              + 1.99777106478e-1) * z2 - 3.33329491539e-1) * z2 * z + z
        ang = (yofs + p) * jnp.float32(57.29577951308232)
        cond = jnp.logical_and(mi_ref[1] > 0, nX > 0.0)
        o_ref[...] = jnp.where(cond, ang, jnp.zeros((8, 128), jnp.float32))


def _tc_regression(sm_cm, meta_i, meta_f):
    out = pl.pallas_call(
        _reg_body,
        grid=(9,),
        in_specs=[
            pl.BlockSpec(memory_space=pltpu.SMEM),
            pl.BlockSpec(memory_space=pltpu.SMEM),
            pl.BlockSpec((_NUM_CLASSES, 128, 128), lambda r: (0, r, 0)),
        ],
        out_specs=pl.BlockSpec((8, 128), lambda r: (0, 0)),
        out_shape=jax.ShapeDtypeStruct((8, 128), jnp.float32),
        scratch_shapes=[pltpu.SMEM((16,), jnp.float32)],
    )(meta_i, meta_f, sm_cm)
    return out[0, 0]


def kernel(cls_score):
    sm_cm = _scores_mean_cm(cls_score)
    smf = sm_cm.reshape(_TOT)
    hist = _sc_hist1(smf)
    ck, cf, cnt = _sc_compact(smf, hist)
    zeros = jnp.zeros((_H2BINS,), jnp.int32)
    meta_i, meta_f = _sc_resolve(hist, ck, cf, cnt, zeros)
    return _tc_regression(sm_cm, meta_i, meta_f).reshape(())


# speedup vs baseline: 11.5095x; 1.3995x over previous
import functools

import jax
import jax.numpy as jnp
from jax import lax
from jax.experimental import pallas as pl
from jax.experimental.pallas import tpu as pltpu
from jax.experimental.pallas import tpu_sc as plsc

_NUM_CLASSES = 15
_THRESH = 0.05
_TOPK = 2000
_N = 384 * 384
_TOT = _N * _NUM_CLASSES

_NW = 32
_CHUNK = _TOT // _NW
_NVEC = _CHUNK // 16
_HBINS = 2048
_L2BINS = 4096
_FBINS = 2048
_CAP = 1024
_BIAS = 0x3D000000
_BIG_FLAT = _TOT + 1


def _mean_body(x_ref, o_ref):
    s = jax.nn.sigmoid(x_ref[...])
    o_ref[...] = jnp.mean(s, axis=0)


def _scores_mean_cm(cls_score):
    x = cls_score.reshape(8, _NUM_CLASSES, 1152, 128)
    return pl.pallas_call(
        _mean_body,
        grid=(_NUM_CLASSES, 3),
        in_specs=[pl.BlockSpec((8, 1, 384, 128), lambda c, r: (0, c, r, 0))],
        out_specs=pl.BlockSpec((1, 384, 128), lambda c, r: (c, r, 0)),
        out_shape=jax.ShapeDtypeStruct((_NUM_CLASSES, 1152, 128), jnp.float32),
    )(x)


def _zero_vmem(ref, nwords):
    z = jnp.zeros((16,), jnp.int32)

    def z_body(i, _):
        ref[pl.ds(i * 16, 16)] = z
        return 0

    lax.fori_loop(0, nwords // 16, z_body, 0)


def _scan_top(hist_ref, nbins, base, target):
    iota = lax.iota(jnp.int32, 16)
    neg = jnp.int32(-2147483647)
    nch = nbins // 16

    def cond(carry):
        found, j, _, _, _ = carry
        return jnp.logical_and(jnp.logical_not(found), j < nch)

    def body(carry):
        _, j, b, cabove, cnt = carry
        c = nch - 1 - j
        h = hist_ref[pl.ds(c * 16, 16)]
        cum = plsc.cumsum(h)
        s = jnp.max(cum)
        ge = (cnt + s) - (cum - h)
        mask = ge >= target
        hit = (cnt + s) >= target
        l_ = jnp.max(jnp.where(mask, iota, -1))
        cum_at = jnp.max(jnp.where(iota == l_, cum, neg))
        nb = c * 16 + l_
        ncab = cnt + s - cum_at
        return (hit, j + 1, jnp.where(hit, nb, b),
                jnp.where(hit, ncab, cabove), cnt + s)

    res = lax.while_loop(cond, body, (jnp.bool_(False), jnp.int32(0),
                                      jnp.int32(-1), base, base))
    return res[0], res[2], res[3]


def _scan_bottom(hist_ref, nbins, target):
    iota = lax.iota(jnp.int32, 16)
    neg = jnp.int32(-2147483647)
    nch = nbins // 16

    def cond(carry):
        found, j, _, _, _ = carry
        return jnp.logical_and(jnp.logical_not(found), j < nch)

    def body(carry):
        _, j, b, cbelow, cnt = carry
        h = hist_ref[pl.ds(j * 16, 16)]
        cum = plsc.cumsum(h)
        s = jnp.max(cum)
        le = cnt + cum
        mask = le >= target
        hit = (cnt + s) >= target
        l_ = jnp.min(jnp.where(mask, iota, 99))
        cum_at = jnp.max(jnp.where(iota == l_, cum, neg))
        h_at = jnp.max(jnp.where(iota == l_, h, neg))
        nb = j * 16 + l_
        ncb = cnt + cum_at - h_at
        return (hit, j + 1, jnp.where(hit, nb, b),
                jnp.where(hit, ncb, cbelow), cnt + s)

    res = lax.while_loop(cond, body, (jnp.bool_(False), jnp.int32(0),
                                      jnp.int32(0), jnp.int32(0), jnp.int32(0)))
    return res[2], res[3]


def _sum_hist(hist_ref, nbins):
    def body(i, acc):
        return acc + hist_ref[pl.ds(i * 16, 16)]

    v = lax.fori_loop(0, nbins // 16, body, jnp.zeros((16,), jnp.int32))
    return jnp.sum(v)


def _merge_hist(hist_hbm_refs, hbuf, H):
    pltpu.sync_copy(hist_hbm_refs, hbuf)

    def m_body(i, _):
        H[pl.ds(i * 16, 16)] = hbuf[0, pl.ds(i * 16, 16)] + hbuf[1, pl.ds(i * 16, 16)]
        return 0

    lax.fori_loop(0, _HBINS // 16, m_body, 0)


_sc_mesh = plsc.VectorSubcoreMesh(core_axis_name="c", subcore_axis_name="s")


@functools.partial(
    pl.kernel,
    out_type=jax.ShapeDtypeStruct((2, _HBINS), jnp.int32),
    mesh=_sc_mesh,
    compiler_params=pltpu.CompilerParams(needs_layout_passes=False),
    scratch_types=[
        pltpu.VMEM((_CHUNK,), jnp.float32),
        pltpu.VMEM((_HBINS,), jnp.int32),
        pltpu.VMEM_SHARED((16, _HBINS), jnp.int32),
        pltpu.VMEM((16, _HBINS), jnp.int32),
    ],
)
def _sc_hist1(smf_hbm, out_hbm, vals, hist, shared, redbuf):
    cid = lax.axis_index("c")
    sid = lax.axis_index("s")
    wid = sid * 2 + cid
    pltpu.sync_copy(smf_hbm.at[pl.ds(wid * _CHUNK, _CHUNK)], vals)
    _zero_vmem(hist, _HBINS)
    ones = jnp.ones((16,), jnp.int32)

    def body(i, _):
        v = vals[pl.ds(i * 16, 16)]
        b = lax.bitcast_convert_type(v, jnp.int32)
        valid = v > _THRESH
        bin_ = lax.shift_right_logical(b - _BIAS, 15)
        bin_ = jnp.where(valid, bin_, 0)
        plsc.addupdate_scatter(hist, [bin_], ones, mask=valid)
        return 0

    lax.fori_loop(0, _NVEC, body, 0, unroll=4)
    pltpu.sync_copy(hist, shared.at[sid])
    plsc.subcore_barrier()

    @pl.when(sid == 0)
    def _():
        pltpu.sync_copy(shared, redbuf)

        def red(i, _):
            acc = redbuf[0, pl.ds(i * 16, 16)]
            for t in range(1, 16):
                acc = acc + redbuf[t, pl.ds(i * 16, 16)]
            hist[pl.ds(i * 16, 16)] = acc
            return 0

        lax.fori_loop(0, _HBINS // 16, red, 0)
        pltpu.sync_copy(hist, out_hbm.at[cid])


@functools.partial(
    pl.kernel,
    out_type=(
        jax.ShapeDtypeStruct((_NW * _CAP,), jnp.int32),
        jax.ShapeDtypeStruct((_NW * _CAP,), jnp.int32),
        jax.ShapeDtypeStruct((_NW * 16,), jnp.int32),
    ),
    mesh=_sc_mesh,
    compiler_params=pltpu.CompilerParams(needs_layout_passes=False),
    scratch_types=[
        pltpu.VMEM((_CHUNK,), jnp.float32),
        pltpu.VMEM((2, _HBINS), jnp.int32),
        pltpu.VMEM((_HBINS,), jnp.int32),
        pltpu.VMEM((_CAP,), jnp.int32),
        pltpu.VMEM((_CAP,), jnp.int32),
        pltpu.VMEM((16,), jnp.int32),
        pltpu.SMEM((4,), jnp.int32),
    ],
)
def _sc_compact(smf_hbm, hist_hbm, ck_hbm, cf_hbm, cnt_hbm,
                vals, hbuf, H, ck, cf, cntv, off_ref):
    cid = lax.axis_index("c")
    sid = lax.axis_index("s")
    wid = sid * 2 + cid
    pltpu.sync_copy(smf_hbm.at[pl.ds(wid * _CHUNK, _CHUNK)], vals)
    _merge_hist(hist_hbm, hbuf, H)
    found, b1, _ = _scan_top(H, _HBINS, jnp.int32(0), jnp.int32(_TOPK))
    b1 = jnp.where(found, b1, jnp.int32(-1))
    iota = lax.iota(jnp.int32, 16)
    off_ref[0] = 0

    def body(i, _):
        v = vals[pl.ds(i * 16, 16)]
        b = lax.bitcast_convert_type(v, jnp.int32)
        key = b - _BIAS
        m = jnp.logical_and(v > _THRESH,
                            lax.shift_right_logical(key, 15) == b1)

        @pl.when(jnp.any(m))
        def _():
            off = off_ref[0]
            incl = plsc.cumsum(m.astype(jnp.int32))
            tot = jnp.max(incl)
            jv = wid * _CHUNK + i * 16 + iota
            cdiv = jv // _N
            flat = (jv - cdiv * _N) * _NUM_CLASSES + cdiv
            pos = off + incl - 1
            m2 = jnp.logical_and(m, pos < _CAP)
            pos = jnp.where(m2, pos, 0)
            plsc.store_scatter(ck, [pos], key, mask=m2)
            plsc.store_scatter(cf, [pos], flat, mask=m2)
            off_ref[0] = off + tot

        return 0

    lax.fori_loop(0, _NVEC, body, 0, unroll=2)
    cntv[...] = jnp.where(iota == 0, jnp.minimum(off_ref[0], _CAP), 0)
    pltpu.sync_copy(ck, ck_hbm.at[pl.ds(wid * _CAP, _CAP)])
    pltpu.sync_copy(cf, cf_hbm.at[pl.ds(wid * _CAP, _CAP)])
    pltpu.sync_copy(cntv, cnt_hbm.at[pl.ds(wid * 16, 16)])


@functools.partial(
    pl.kernel,
    out_type=(
        jax.ShapeDtypeStruct((16,), jnp.int32),
        jax.ShapeDtypeStruct((16,), jnp.float32),
    ),
    mesh=_sc_mesh,
    compiler_params=pltpu.CompilerParams(needs_layout_passes=False),
    scratch_types=[
        pltpu.VMEM((2, _HBINS), jnp.int32),
        pltpu.VMEM((_HBINS,), jnp.int32),
        pltpu.VMEM((_NW * _CAP,), jnp.int32),
        pltpu.VMEM((_NW * _CAP,), jnp.int32),
        pltpu.VMEM((_L2BINS,), jnp.int32),
        pltpu.VMEM((16,), jnp.int32),
        pltpu.VMEM((_FBINS,), jnp.int32),
        pltpu.VMEM((_NW * 16,), jnp.int32),
        pltpu.VMEM((16,), jnp.int32),
        pltpu.VMEM((16,), jnp.float32),
    ],
)
def _sc_resolve(hist_hbm, ck_hbm, cf_hbm, cnt_hbm, zeros_hbm,
                mi_hbm, mf_hbm,
                hbuf, H, keys, flats, hist2, histb, histf, cnts, miv, mfv):
    cid = lax.axis_index("c")
    sid = lax.axis_index("s")
    wid = sid * 2 + cid

    @pl.when(wid == 0)
    def _():
        _merge_hist(hist_hbm, hbuf, H)
        num_valid = _sum_hist(H, _HBINS)
        found, b1, cabove1 = _scan_top(
            H, _HBINS, jnp.int32(0), jnp.int32(_TOPK))
        pltpu.sync_copy(cnt_hbm, cnts)
        pltpu.sync_copy(ck_hbm, keys)
        pltpu.sync_copy(cf_hbm, flats)
        pltpu.sync_copy(zeros_hbm, hist2)
        ones = jnp.ones((16,), jnp.int32)
        iota = lax.iota(jnp.int32, 16)

        def _for_cands(fn):
            def t_body(t, _):
                cnt_t = cnts[pl.ds(t * 16, 16)][0]

                def i_body(i, _):
                    base = t * _CAP + i * 16
                    k = keys[pl.ds(base, 16)]
                    f = flats[pl.ds(base, 16)]
                    msk = (i * 16 + iota) < cnt_t
                    fn(k, f, msk)
                    return 0

                lax.fori_loop(0, (cnt_t + 15) // 16, i_body, 0)
                return 0

            lax.fori_loop(0, _NW, t_body, 0)

        def p1(k, f, msk):
            bin_ = jnp.where(msk, lax.shift_right_logical(
                jnp.bitwise_and(k, 0x7FFF), 3), 0)
            plsc.addupdate_scatter(hist2, [bin_], ones, mask=msk)

        _for_cands(p1)
        _, b2a, cabove2 = _scan_top(hist2, _L2BINS, cabove1, jnp.int32(_TOPK))
        p12 = b1 * 4096 + jnp.maximum(b2a, 0)

        histb[...] = jnp.zeros((16,), jnp.int32)

        def p2(k, f, msk):
            m = jnp.logical_and(msk, lax.shift_right_logical(k, 3) == p12)
            bin_ = jnp.where(m, jnp.bitwise_and(k, 7), 0)
            plsc.addupdate_scatter(histb, [bin_], ones, mask=m)

        _for_cands(p2)
        _, b3, count_gt = _scan_top(histb, 16, cabove2, jnp.int32(_TOPK))
        v26 = p12 * 8 + jnp.maximum(b3, 0)
        r = jnp.int32(_TOPK) - count_gt

        pltpu.sync_copy(zeros_hbm.at[pl.ds(0, _FBINS)], histf)

        def p3(k, f, msk):
            m = jnp.logical_and(msk, k == v26)
            bin_ = jnp.where(m, lax.shift_right_logical(f, 11), 0)
            plsc.addupdate_scatter(histf, [bin_], ones, mask=m)

        _for_cands(p3)
        bf1, cbelow1 = _scan_bottom(histf, _FBINS, r)
        r2 = r - cbelow1

        pltpu.sync_copy(zeros_hbm.at[pl.ds(0, _FBINS)], histf)

        def p4(k, f, msk):
            m = jnp.logical_and(msk, k == v26)
            m = jnp.logical_and(m, lax.shift_right_logical(f, 11) == bf1)
            bin_ = jnp.where(m, jnp.bitwise_and(f, 0x7FF), 0)
            plsc.addupdate_scatter(histf, [bin_], ones, mask=m)

        _for_cands(p4)
        bf2, _ = _scan_bottom(histf, _FBINS, r2)
        istar = bf1 * 2048 + bf2

        thresh_bits = jnp.int32(0x3D4CCCCD)
        vbits = jnp.where(found, v26 + _BIAS, thresh_bits)
        istar = jnp.where(found, istar, jnp.int32(-1))
        valid_any = (num_valid > 0).astype(jnp.int32)

        mi = (jnp.where(iota == 0, istar, 0)
              + jnp.where(iota == 1, valid_any, 0)
              + jnp.where(iota == 2, found.astype(jnp.int32), 0)
              + jnp.where(iota == 3, num_valid, 0))
        miv[...] = mi
        mfv[...] = lax.bitcast_convert_type(
            jnp.broadcast_to(vbits, (16,)), jnp.float32)
        pltpu.sync_copy(miv, mi_hbm)
        pltpu.sync_copy(mfv, mf_hbm)


def _reg_body(mi_ref, mf_ref, x_ref, o_ref, acc_ref):
    r = pl.program_id(0)

    @pl.when(r == 0)
    def _():
        for i in range(12):
            acc_ref[i] = 0.0

    V = mf_ref[0]
    istar = mi_ref[0]
    blk = x_ref[...]
    row2d = lax.broadcasted_iota(jnp.int32, (128, 128), 0)
    lane2d = lax.broadcasted_iota(jnp.int32, (128, 128), 1)
    n = (r * 128 + row2d) * 128 + lane2d
    keep = None
    for c in range(_NUM_CLASSES):
        v = blk[c]
        flat = n * _NUM_CLASSES + c
        sel = jnp.logical_or(
            v > V, jnp.logical_and(v == V, flat <= istar))
        keep = sel if keep is None else jnp.logical_or(keep, sel)
    x = blk[3]
    y = blk[5]
    lab = x > y
    fx = jnp.logical_and(keep, lab).astype(jnp.float32)
    fy = jnp.logical_and(keep, jnp.logical_not(lab)).astype(jnp.float32)
    acc_ref[0] += jnp.sum(fx)
    acc_ref[1] += jnp.sum(fx * x)
    acc_ref[2] += jnp.sum(fx * y)
    acc_ref[3] += jnp.sum(fx * x * x)
    acc_ref[4] += jnp.sum(fx * x * y)
    acc_ref[5] += jnp.sum(fy)
    acc_ref[6] += jnp.sum(fy * x)
    acc_ref[7] += jnp.sum(fy * y)
    acc_ref[8] += jnp.sum(fy * x * x)
    acc_ref[9] += jnp.sum(fy * x * y)

    @pl.when(r == 8)
    def _():
        nX = acc_ref[0]
        sxX, syX, sxxX, sxyX = acc_ref[1], acc_ref[2], acc_ref[3], acc_ref[4]
        nY = acc_ref[5]
        sxY, syY, sxxY, sxyY = acc_ref[6], acc_ref[7], acc_ref[8], acc_ref[9]
        slope_x = (sxyX - sxX * syX / nX) / (sxxX - sxX * sxX / nX)
        slope_y = (sxyY - sxY * syY / nY) / (sxxY - sxY * sxY / nY)
        t = jnp.abs((slope_y - slope_x) / (1.0 + slope_y * slope_x + 1e-05))
        tv = jnp.full((8, 128), t)
        hi = tv > 2.414213562373095
        mid = tv > 0.414213562373095
        yofs = jnp.where(hi, jnp.float32(1.5707963267948966),
                         jnp.where(mid, jnp.float32(0.7853981633974483), 0.0))
        z = jnp.where(hi, -1.0 / tv,
                      jnp.where(mid, (tv - 1.0) / (tv + 1.0), tv))
        z2 = z * z
        p = (((8.05374449538e-2 * z2 - 1.38776856032e-1) * z2
              + 1.99777106478e-1) * z2 - 3.33329491539e-1) * z2 * z + z
        ang = (yofs + p) * jnp.float32(57.29577951308232)
        cond = jnp.logical_and(mi_ref[1] > 0, nX > 0.0)
        o_ref[...] = jnp.where(cond, ang, jnp.zeros((8, 128), jnp.float32))


def _tc_regression(sm_cm, meta_i, meta_f):
    out = pl.pallas_call(
        _reg_body,
        grid=(9,),
        in_specs=[
            pl.BlockSpec(memory_space=pltpu.SMEM),
            pl.BlockSpec(memory_space=pltpu.SMEM),
            pl.BlockSpec((_NUM_CLASSES, 128, 128), lambda r: (0, r, 0)),
        ],
        out_specs=pl.BlockSpec((8, 128), lambda r: (0, 0)),
        out_shape=jax.ShapeDtypeStruct((8, 128), jnp.float32),
        scratch_shapes=[pltpu.SMEM((16,), jnp.float32)],
    )(meta_i, meta_f, sm_cm)
    return out[0, 0]


def kernel(cls_score):
    sm_cm = _scores_mean_cm(cls_score)
    smf = sm_cm.reshape(_TOT)
    hist = _sc_hist1(smf)
    ck, cf, cnt = _sc_compact(smf, hist)
    zeros = jnp.zeros((_L2BINS,), jnp.int32)
    meta_i, meta_f = _sc_resolve(hist, ck, cf, cnt, zeros)
    return _tc_regression(sm_cm, meta_i, meta_f).reshape(())


# speedup vs baseline: 17.7441x; 1.5417x over previous
import functools

import jax
import jax.numpy as jnp
from jax import lax
from jax.experimental import pallas as pl
from jax.experimental.pallas import tpu as pltpu
from jax.experimental.pallas import tpu_sc as plsc

_NUM_CLASSES = 15
_THRESH = 0.05
_TOPK = 2000
_N = 384 * 384
_TOT = _N * _NUM_CLASSES

_NW = 32
_CHUNK = _TOT // _NW
_NVEC = _CHUNK // 16
_HBINS = 2048
_L2BINS = 4096
_FBINS = 2048
_CAP = 1024
_BIAS = 0x3D000000
_BIG_FLAT = _TOT + 1


def _mean_body(x_ref, o_ref):
    s = jax.nn.sigmoid(x_ref[...])
    o_ref[...] = jnp.mean(s, axis=0)


def _scores_mean_cm(cls_score):
    x = cls_score.reshape(8, _NUM_CLASSES, 1152, 128)
    return pl.pallas_call(
        _mean_body,
        grid=(_NUM_CLASSES, 3),
        in_specs=[pl.BlockSpec((8, 1, 384, 128), lambda c, r: (0, c, r, 0))],
        out_specs=pl.BlockSpec((1, 384, 128), lambda c, r: (c, r, 0)),
        out_shape=jax.ShapeDtypeStruct((_NUM_CLASSES, 1152, 128), jnp.float32),
    )(x)


def _zero_vmem(ref, nwords):
    z = jnp.zeros((16,), jnp.int32)

    def z_body(i, _):
        ref[pl.ds(i * 16, 16)] = z
        return 0

    lax.fori_loop(0, nwords // 16, z_body, 0)


def _scan_top(hist_ref, nbins, base, target):
    iota = lax.iota(jnp.int32, 16)
    neg = jnp.int32(-2147483647)
    nch = nbins // 16

    def cond(carry):
        found, j, _, _, _ = carry
        return jnp.logical_and(jnp.logical_not(found), j < nch)

    def body(carry):
        _, j, b, cabove, cnt = carry
        c = nch - 1 - j
        h = hist_ref[pl.ds(c * 16, 16)]
        cum = plsc.cumsum(h)
        s = jnp.max(cum)
        ge = (cnt + s) - (cum - h)
        mask = ge >= target
        hit = (cnt + s) >= target
        l_ = jnp.max(jnp.where(mask, iota, -1))
        cum_at = jnp.max(jnp.where(iota == l_, cum, neg))
        nb = c * 16 + l_
        ncab = cnt + s - cum_at
        return (hit, j + 1, jnp.where(hit, nb, b),
                jnp.where(hit, ncab, cabove), cnt + s)

    res = lax.while_loop(cond, body, (jnp.bool_(False), jnp.int32(0),
                                      jnp.int32(-1), base, base))
    return res[0], res[2], res[3]


def _scan_bottom(hist_ref, nbins, target):
    iota = lax.iota(jnp.int32, 16)
    neg = jnp.int32(-2147483647)
    nch = nbins // 16

    def cond(carry):
        found, j, _, _, _ = carry
        return jnp.logical_and(jnp.logical_not(found), j < nch)

    def body(carry):
        _, j, b, cbelow, cnt = carry
        h = hist_ref[pl.ds(j * 16, 16)]
        cum = plsc.cumsum(h)
        s = jnp.max(cum)
        le = cnt + cum
        mask = le >= target
        hit = (cnt + s) >= target
        l_ = jnp.min(jnp.where(mask, iota, 99))
        cum_at = jnp.max(jnp.where(iota == l_, cum, neg))
        h_at = jnp.max(jnp.where(iota == l_, h, neg))
        nb = j * 16 + l_
        ncb = cnt + cum_at - h_at
        return (hit, j + 1, jnp.where(hit, nb, b),
                jnp.where(hit, ncb, cbelow), cnt + s)

    res = lax.while_loop(cond, body, (jnp.bool_(False), jnp.int32(0),
                                      jnp.int32(0), jnp.int32(0), jnp.int32(0)))
    return res[2], res[3]


def _sum_hist(hist_ref, nbins):
    def body(i, acc):
        return acc + hist_ref[pl.ds(i * 16, 16)]

    v = lax.fori_loop(0, nbins // 16, body, jnp.zeros((16,), jnp.int32))
    return jnp.sum(v)


def _merge_hist(hist_hbm_refs, hbuf, H):
    pltpu.sync_copy(hist_hbm_refs, hbuf)

    def m_body(i, _):
        H[pl.ds(i * 16, 16)] = hbuf[0, pl.ds(i * 16, 16)] + hbuf[1, pl.ds(i * 16, 16)]
        return 0

    lax.fori_loop(0, _HBINS // 16, m_body, 0)


_sc_mesh = plsc.VectorSubcoreMesh(core_axis_name="c", subcore_axis_name="s")


@functools.partial(
    pl.kernel,
    out_type=(
        jax.ShapeDtypeStruct((2, _HBINS), jnp.int32),
        jax.ShapeDtypeStruct((_NW * _NVEC,), jnp.int32),
    ),
    mesh=_sc_mesh,
    compiler_params=pltpu.CompilerParams(needs_layout_passes=False),
    scratch_types=[
        pltpu.VMEM((_CHUNK,), jnp.float32),
        pltpu.VMEM((_HBINS * 8,), jnp.int32),
        pltpu.VMEM((_HBINS,), jnp.int32),
        pltpu.VMEM((_NVEC,), jnp.int32),
        pltpu.VMEM_SHARED((16, _HBINS), jnp.int32),
        pltpu.VMEM((16, _HBINS), jnp.int32),
    ],
)
def _sc_hist1(smf_hbm, out_hbm, gmax_hbm, vals, hist8, hist, gmax, shared, redbuf):
    cid = lax.axis_index("c")
    sid = lax.axis_index("s")
    wid = sid * 2 + cid
    pltpu.sync_copy(smf_hbm.at[pl.ds(wid * _CHUNK, _CHUNK)], vals)
    _zero_vmem(hist8, _HBINS * 8)
    ones = jnp.ones((16,), jnp.int32)
    iota = lax.iota(jnp.int32, 16)
    lane8 = jnp.bitwise_and(iota, 7)
    neg = jnp.full((16,), -2147483647, jnp.int32)

    def g_body(g, _):
        gmv = neg
        for t in range(16):
            i = g * 16 + t
            v = vals[pl.ds(i * 16, 16)]
            b = lax.bitcast_convert_type(v, jnp.int32)
            valid = v > _THRESH
            key = b - _BIAS
            bin_ = lax.shift_right_logical(key, 15)
            bin8 = jnp.where(valid, bin_ * 8 + lane8, lane8)
            plsc.addupdate_scatter(hist8, [bin8], ones, mask=valid)
            gmv = jnp.maximum(gmv, jnp.where(valid, key, neg))
        gmax[pl.ds(g * 16, 16)] = gmv
        return 0

    lax.fori_loop(0, _NVEC // 16, g_body, 0)
    pltpu.sync_copy(gmax, gmax_hbm.at[pl.ds(wid * _NVEC, _NVEC)])

    def fold2_body(i, _):
        base = i * 128
        acc = plsc.load_gather(hist8, [base + iota * 8])
        for t in range(1, 8):
            acc = acc + plsc.load_gather(hist8, [base + iota * 8 + t])
        hist[pl.ds(i * 16, 16)] = acc
        return 0

    lax.fori_loop(0, _HBINS // 16, fold2_body, 0)
    pltpu.sync_copy(hist, shared.at[sid])
    plsc.subcore_barrier()

    @pl.when(sid == 0)
    def _():
        pltpu.sync_copy(shared, redbuf)

        def red(i, _):
            acc = redbuf[0, pl.ds(i * 16, 16)]
            for t in range(1, 16):
                acc = acc + redbuf[t, pl.ds(i * 16, 16)]
            hist[pl.ds(i * 16, 16)] = acc
            return 0

        lax.fori_loop(0, _HBINS // 16, red, 0)
        pltpu.sync_copy(hist, out_hbm.at[cid])


@functools.partial(
    pl.kernel,
    out_type=(
        jax.ShapeDtypeStruct((_NW * _CAP,), jnp.int32),
        jax.ShapeDtypeStruct((_NW * _CAP,), jnp.int32),
        jax.ShapeDtypeStruct((_NW * 128,), jnp.int32),
        jax.ShapeDtypeStruct((16,), jnp.int32),
    ),
    mesh=_sc_mesh,
    compiler_params=pltpu.CompilerParams(needs_layout_passes=False),
    scratch_types=[
        pltpu.VMEM((_CHUNK,), jnp.float32),
        pltpu.VMEM((2, _HBINS), jnp.int32),
        pltpu.VMEM((_HBINS,), jnp.int32),
        pltpu.VMEM((_NVEC,), jnp.int32),
        pltpu.VMEM((_CAP,), jnp.int32),
        pltpu.VMEM((_CAP,), jnp.int32),
        pltpu.VMEM((128,), jnp.int32),
        pltpu.VMEM((16,), jnp.int32),
        pltpu.SMEM((4,), jnp.int32),
    ],
)
def _sc_compact(smf_hbm, gmax_hbm, hist_hbm, ck_hbm, cf_hbm, cnt_hbm, mb_hbm,
                vals, hbuf, H, gmax, ck, cf, cntv, mbv, off_ref):
    cid = lax.axis_index("c")
    sid = lax.axis_index("s")
    wid = sid * 2 + cid
    pltpu.sync_copy(smf_hbm.at[pl.ds(wid * _CHUNK, _CHUNK)], vals)
    pltpu.sync_copy(gmax_hbm.at[pl.ds(wid * _NVEC, _NVEC)], gmax)
    _merge_hist(hist_hbm, hbuf, H)
    found, b1, cabove1 = _scan_top(H, _HBINS, jnp.int32(0), jnp.int32(_TOPK))
    b1 = jnp.where(found, b1, jnp.int32(-1))
    thr_lo = b1 * 32768
    iota = lax.iota(jnp.int32, 16)
    off_ref[0] = 0

    def g_body(g, _):
        gv = gmax[pl.ds(g * 16, 16)]
        m0 = gv >= thr_lo

        @pl.when(jnp.any(m0))
        def _():
            def col_cond(carry):
                m, _ = carry
                return jnp.any(m > 0)

            def col_body(carry):
                m, off = carry
                l = plsc.all_reduce_ffs(m > 0)[0]
                idx = g * 256 + l + iota * 16
                vv = plsc.load_gather(vals, [idx])
                b = lax.bitcast_convert_type(vv, jnp.int32)
                key = b - _BIAS
                mm = jnp.logical_and(
                    vv > _THRESH,
                    lax.shift_right_logical(key, 15) == b1)
                incl = plsc.cumsum(mm.astype(jnp.int32))
                tot = jnp.max(incl)
                jv = wid * _CHUNK + idx
                cdiv = jv // _N
                flat = (jv - cdiv * _N) * _NUM_CLASSES + cdiv
                pos = off + incl - 1
                m2 = jnp.logical_and(mm, pos < _CAP)
                pos = jnp.where(m2, pos, 0)
                plsc.store_scatter(ck, [pos], key, mask=m2)
                plsc.store_scatter(cf, [pos], flat, mask=m2)
                return jnp.where(iota != l, m, 0), off + tot

            m_end, off_end = lax.while_loop(
                col_cond, col_body, (m0.astype(jnp.int32), off_ref[0]))
            off_ref[0] = off_end

        return 0

    lax.fori_loop(0, _NVEC // 16, g_body, 0)
    cnt_final = jnp.minimum(off_ref[0], _CAP)

    def c_body(i, _):
        cntv[pl.ds(i * 16, 16)] = jnp.broadcast_to(cnt_final, (16,))
        return 0

    lax.fori_loop(0, 8, c_body, 0)
    pltpu.sync_copy(ck, ck_hbm.at[pl.ds(wid * _CAP, _CAP)])
    pltpu.sync_copy(cf, cf_hbm.at[pl.ds(wid * _CAP, _CAP)])
    pltpu.sync_copy(cntv, cnt_hbm.at[pl.ds(wid * 128, 128)])

    @pl.when(wid == 0)
    def _():
        num_valid = _sum_hist(H, _HBINS)
        foundi = jnp.int32(1) - jnp.where(found, 0, 1)
        is0 = (iota == 0).astype(jnp.int32)
        is1 = (iota == 1).astype(jnp.int32)
        is2 = (iota == 2).astype(jnp.int32)
        is3 = (iota == 3).astype(jnp.int32)
        mbv[...] = (is0 * foundi + is1 * jnp.maximum(b1, 0)
                    + is2 * cabove1 + is3 * num_valid)
        pltpu.sync_copy(mbv, mb_hbm)


def _reg_body(mb_ref, x_ref, ck_ref, cf_ref, cnt_ref, o_ref, acc_ref, si_ref):
    r = pl.program_id(0)

    @pl.when(r == 0)
    def _():
        for i in range(12):
            acc_ref[i] = 0.0
        found = mb_ref[0] > 0
        b1 = mb_ref[1]
        cabove1 = mb_ref[2]
        num_valid = mb_ref[3]
        kb = ck_ref[...]
        fb = cf_ref[...]
        cnts = cnt_ref[...][:, 0:1]
        pos = lax.broadcasted_iota(jnp.int32, (_NW, _CAP), 1)
        maskc = pos < cnts
        k_t = jnp.float32(_TOPK) - cabove1.astype(jnp.float32)

        def fcnt(x):
            return jnp.sum(jnp.where(
                jnp.logical_and(maskc, kb >= x), 1.0, 0.0))

        def v_body(_, lohi):
            lo, hi = lohi
            mid = lax.shift_right_logical(lo + hi + 1, 1)
            take = fcnt(mid) >= k_t
            return jnp.where(take, mid, lo), jnp.where(take, hi, mid - 1)

        lo0 = b1 * 32768
        hi0 = lo0 + 32767
        v26, _ = lax.fori_loop(0, 15, v_body, (lo0, hi0))
        count_gt = cabove1.astype(jnp.float32) + fcnt(v26 + 1)
        rr = jnp.float32(_TOPK) - count_gt
        eqmask = jnp.logical_and(maskc, kb == v26)

        def gcnt(x):
            return jnp.sum(jnp.where(
                jnp.logical_and(eqmask, fb <= x), 1.0, 0.0))

        def t_body(_, lohi):
            lo, hi = lohi
            mid = lax.shift_right_logical(lo + hi, 1)
            take = gcnt(mid) >= rr
            return jnp.where(take, lo, mid + 1), jnp.where(take, mid, hi)

        istar, _ = lax.fori_loop(0, 22, t_body,
                                 (jnp.int32(0), jnp.int32((1 << 22) - 1)))
        thresh_key = jnp.int32(0x3D4CCCCD - _BIAS)
        si_ref[0] = jnp.where(found, v26, thresh_key)
        si_ref[1] = jnp.where(found, istar, jnp.int32(-1))
        si_ref[2] = (num_valid > 0).astype(jnp.int32)

    v26s = si_ref[0]
    istar = si_ref[1]
    blk = x_ref[...]
    row2d = lax.broadcasted_iota(jnp.int32, (128, 128), 0)
    lane2d = lax.broadcasted_iota(jnp.int32, (128, 128), 1)
    n = (r * 128 + row2d) * 128 + lane2d
    keep = None
    for c in range(_NUM_CLASSES):
        key_c = lax.bitcast_convert_type(blk[c], jnp.int32) - _BIAS
        flat = n * _NUM_CLASSES + c
        sel = jnp.logical_or(
            key_c > v26s, jnp.logical_and(key_c == v26s, flat <= istar))
        keep = sel if keep is None else jnp.logical_or(keep, sel)
    x = blk[3]
    y = blk[5]
    lab = x > y
    fx = jnp.logical_and(keep, lab).astype(jnp.float32)
    fy = jnp.logical_and(keep, jnp.logical_not(lab)).astype(jnp.float32)
    acc_ref[0] += jnp.sum(fx)
    acc_ref[1] += jnp.sum(fx * x)
    acc_ref[2] += jnp.sum(fx * y)
    acc_ref[3] += jnp.sum(fx * x * x)
    acc_ref[4] += jnp.sum(fx * x * y)
    acc_ref[5] += jnp.sum(fy)
    acc_ref[6] += jnp.sum(fy * x)
    acc_ref[7] += jnp.sum(fy * y)
    acc_ref[8] += jnp.sum(fy * x * x)
    acc_ref[9] += jnp.sum(fy * x * y)

    @pl.when(r == 8)
    def _():
        nX = acc_ref[0]
        sxX, syX, sxxX, sxyX = acc_ref[1], acc_ref[2], acc_ref[3], acc_ref[4]
        nY = acc_ref[5]
        sxY, syY, sxxY, sxyY = acc_ref[6], acc_ref[7], acc_ref[8], acc_ref[9]
        slope_x = (sxyX - sxX * syX / nX) / (sxxX - sxX * sxX / nX)
        slope_y = (sxyY - sxY * syY / nY) / (sxxY - sxY * sxY / nY)
        t = jnp.abs((slope_y - slope_x) / (1.0 + slope_y * slope_x + 1e-05))
        tv = jnp.full((8, 128), t)
        hi = tv > 2.414213562373095
        mid = tv > 0.414213562373095
        yofs = jnp.where(hi, jnp.float32(1.5707963267948966),
                         jnp.where(mid, jnp.float32(0.7853981633974483), 0.0))
        z = jnp.where(hi, -1.0 / tv,
                      jnp.where(mid, (tv - 1.0) / (tv + 1.0), tv))
        z2 = z * z
        p = (((8.05374449538e-2 * z2 - 1.38776856032e-1) * z2
              + 1.99777106478e-1) * z2 - 3.33329491539e-1) * z2 * z + z
        ang = (yofs + p) * jnp.float32(57.29577951308232)
        cond = jnp.logical_and(si_ref[2] > 0, nX > 0.0)
        o_ref[...] = jnp.where(cond, ang, jnp.zeros((8, 128), jnp.float32))


def _tc_regression(sm_cm, meta_b, ck, cf, cnt):
    out = pl.pallas_call(
        _reg_body,
        grid=(9,),
        in_specs=[
            pl.BlockSpec(memory_space=pltpu.SMEM),
            pl.BlockSpec((_NUM_CLASSES, 128, 128), lambda r: (0, r, 0)),
            pl.BlockSpec((_NW, _CAP), lambda r: (0, 0)),
            pl.BlockSpec((_NW, _CAP), lambda r: (0, 0)),
            pl.BlockSpec((_NW, 128), lambda r: (0, 0)),
        ],
        out_specs=pl.BlockSpec((8, 128), lambda r: (0, 0)),
        out_shape=jax.ShapeDtypeStruct((8, 128), jnp.float32),
        scratch_shapes=[pltpu.SMEM((16,), jnp.float32),
                        pltpu.SMEM((8,), jnp.int32)],
    )(meta_b, sm_cm, ck.reshape(_NW, _CAP), cf.reshape(_NW, _CAP),
      cnt.reshape(_NW, 128))
    return out[0, 0]


def kernel(cls_score):
    sm_cm = _scores_mean_cm(cls_score)
    smf = sm_cm.reshape(_TOT)
    hist, gmax = _sc_hist1(smf)
    ck, cf, cnt, meta_b = _sc_compact(smf, gmax, hist)
    return _tc_regression(sm_cm, meta_b, ck, cf, cnt).reshape(())
